# baseline (speedup 1.0000x reference)
import functools

import jax
import jax.numpy as jnp
from jax import lax
from jax.experimental import pallas as pl
from jax.experimental.pallas import tpu as pltpu
from jax.experimental.pallas import tpu_sc as plsc

BATCH = 1024
D = 26
M = 4
R = 2048
C = 16
DM = D * M
JC = (D + 1) * C
K = max(1, int(0.1 * R))

BB = 256
GRID = BATCH // BB

NC = 2
NS = 16
NW = NC * NS
RW = BATCH // NW
RV = R // 16


def _tc1_body(xr_ref, cf_ref, wf_ref, rules_ref, firing_ref, s_scr):
    @pl.when(pl.program_id(0) == 0)
    def _():
        mcol = lax.broadcasted_iota(jnp.int32, (R, DM), 1) % M
        s_scr[...] = (rules_ref[...] == mcol).astype(jnp.float32)

    wf = wf_ref[...]
    inv = 1.0 / (2.0 * wf * wf)
    dq = xr_ref[...] - cf_ref[...]
    q = dq * dq * inv
    logits = lax.dot_general(q, s_scr[...], (((1,), (1,)), ((), ())),
                             preferred_element_type=jnp.float32,
                             precision=lax.Precision.HIGHEST)
    firing_ref[...] = lax.bitcast_convert_type(jnp.exp(-logits), jnp.int32)


def _sel(vec, idx):
    lane = jnp.arange(16, dtype=jnp.int32)
    return jnp.sum(jnp.where(lane == idx, vec, 0))


def _sc_body(firing_hbm, kth_hbm, fbuf, hist, histc, cand, kbuf):
    rw = fbuf.shape[0]
    wid = lax.axis_index("s") * NC + lax.axis_index("c")
    base = wid * rw
    with jax.named_scope("sc_dma_in"):
        pltpu.sync_copy(firing_hbm.at[pl.ds(base, rw)], fbuf)

    lane = jnp.arange(16, dtype=jnp.int32)
    zeros16 = jnp.zeros((16,), jnp.int32)
    ones16 = jnp.ones((16,), jnp.int32)

    @plsc.parallel_loop(0, R // 16, unroll=8)
    def _zero(i):
        hist[pl.ds(i * 16, 16)] = zeros16

    def row_body(i, _):
        lbase = lane * 128

        with jax.named_scope("sc_hist"):
            @plsc.parallel_loop(0, RV, unroll=8)
            def _hist(j):
                u = fbuf[i, pl.ds(j * 16, 16)]
                b = lax.shift_right_logical(u, 23)
                plsc.addupdate_scatter(hist, [lbase + b], ones16)

        sc_combine = jax.named_scope("sc_combine"); sc_combine.__enter__()
        t_vec = zeros16
        for c in range(8):
            acc = zeros16
            for l in range(16):
                off = l * 128 + c * 16
                acc = acc + hist[pl.ds(off, 16)]
                hist[pl.ds(off, 16)] = zeros16
            histc[pl.ds(c * 16, 16)] = acc
            t_vec = jnp.where(lane == 8 + c, jnp.sum(acc), t_vec)

        tr = lax.rev(t_vec, (0,))
        ct = plsc.cumsum(tr)
        a_exc = ct - tr
        i_star = plsc.all_reduce_ffs(ct >= K)
        i_s = _sel(lane, i_star)
        c_star = 7 - i_s
        cum_prev = _sel(a_exc, i_star)
        acc_star = histc[pl.ds(c_star * 16, 16)]
        racc = lax.rev(acc_star, (0,))
        pcl = plsc.cumsum(racc)
        j_star = plsc.all_reduce_ffs(cum_prev + pcl >= K)
        j_s = _sel(lane, j_star)
        e1 = c_star * 16 + 15 - j_s
        count_ge = cum_prev + _sel(pcl, j_star)
        m_e1 = _sel(racc, j_star)
        rem = K - (count_ge - m_e1)
        sc_combine.__exit__(None, None, None)

        with jax.named_scope("sc_compact"):
            @plsc.parallel_loop(0, RV, unroll=4, carry=zeros16)
            def offv(j, ov):
                u = fbuf[i, pl.ds(j * 16, 16)]
                msk = lax.shift_right_logical(u, 23) == e1
                mi = msk.astype(jnp.int32)
                pos = plsc.cumsum(mi) - mi
                plsc.store_scatter(cand, [ov + pos], u, mask=msk)
                return ov + plsc.all_reduce_population_count(msk)
        m = jnp.sum(jnp.where(lane == 0, offv, 0))
        cand[pl.ds(m, 16)] = zeros16
        nv = lax.shift_right_logical(m + 15, 4)

        ebits = lax.shift_left(e1, 23)

        def bit_body(bi, t):
            t_try = jnp.bitwise_or(t, lax.shift_left(jnp.int32(1), 22 - bi))
            full = jnp.bitwise_or(ebits, t_try)

            def cnt_body(j, cv):
                u = cand[pl.ds(j * 16, 16)]
                return cv + (u >= full).astype(jnp.int32)

            cv = lax.fori_loop(0, nv, cnt_body, zeros16)
            return jnp.where(jnp.sum(cv) >= rem, t_try, t)

        with jax.named_scope("sc_binsearch"):
            t_fin = lax.fori_loop(0, 23, bit_body, jnp.int32(0))
        kf = jnp.bitwise_or(ebits, t_fin)
        plsc.store_scatter(kbuf, [jnp.full((16,), i, jnp.int32)],
                           jnp.full((16,), kf, jnp.int32),
                           mask=lane == 0)
        return 0

    lax.fori_loop(0, rw, row_body, 0)
    pltpu.sync_copy(kbuf, kth_hbm.at[pl.ds(base, rw)])


def _sc_select(firing_bits, interpret=False):
    nb = firing_bits.shape[0]
    rw = nb // NW
    mesh = plsc.VectorSubcoreMesh(core_axis_name="c", subcore_axis_name="s",
                                  num_cores=NC, num_subcores=NS)
    return pl.kernel(
        _sc_body,
        out_type=jax.ShapeDtypeStruct((nb,), jnp.int32),
        mesh=mesh,
        interpret=interpret,
        compiler_params=pltpu.CompilerParams(needs_layout_passes=False),
        scratch_types=[
            pltpu.VMEM((rw, R), jnp.int32),
            pltpu.VMEM((R,), jnp.int32),
            pltpu.VMEM((128,), jnp.int32),
            pltpu.VMEM((R + 32,), jnp.int32),
            pltpu.VMEM((rw,), jnp.int32),
        ],
    )(firing_bits)


def _tc2_body(x_ref, f1_ref, f2_ref, k1_ref, k2_ref, cons_ref,
              out_ref, norm_ref, mask_ref, cs_scr):
    @pl.when(pl.program_id(0) == 0)
    def _():
        p = (lax.broadcasted_iota(jnp.int32, (JC, C), 0) % C
             == lax.broadcasted_iota(jnp.int32, (JC, C), 1)).astype(jnp.float32)
        cs_scr[...] = lax.dot_general(cons_ref[...], p,
                                      (((1,), (0,)), ((), ())),
                                      preferred_element_type=jnp.float32,
                                      precision=lax.Precision.DEFAULT)

    lo = pl.program_id(0) < GRID // 2
    fi = jnp.where(lo, f1_ref[...], f2_ref[...])
    kth = jnp.where(lo, k1_ref[...], k2_ref[...])
    firing = lax.bitcast_convert_type(fi, jnp.float32)
    maskf = (fi >= kth).astype(jnp.float32)
    fm = firing * maskf
    denom = jnp.sum(fm, axis=1, keepdims=True) + 1e-9
    normv = fm / denom
    s_ext = jnp.sum(x_ref[...], axis=1, keepdims=True) + 1.0
    outv = s_ext * lax.dot_general(normv, cs_scr[...],
                                   (((1,), (0,)), ((), ())),
                                   preferred_element_type=jnp.float32,
                                   precision=lax.Precision.DEFAULT)
    out_ref[...] = outv
    norm_ref[...] = normv
    mask_ref[...] = maskf


@functools.partial(jax.jit, static_argnames=("interpret",))
def kernel(x, centers, widths, consequents, rules, interpret=False):
    x = x.astype(jnp.float32)
    xr = jnp.repeat(x, M, axis=1)
    cf = centers.astype(jnp.float32).reshape(1, DM)
    wf = widths.astype(jnp.float32).reshape(1, DM)
    rules_rep = jnp.repeat(rules.astype(jnp.int32), M, axis=1)
    cons2 = consequents.astype(jnp.float32).reshape(R, JC)

    halves = []
    for h in range(2):
        fb = pl.pallas_call(
            _tc1_body,
            grid=(GRID // 2,),
            in_specs=[
                pl.BlockSpec((BB, DM), lambda i, h=h: (GRID // 2 * h + i, 0)),
                pl.BlockSpec((1, DM), lambda i: (0, 0)),
                pl.BlockSpec((1, DM), lambda i: (0, 0)),
                pl.BlockSpec((R, DM), lambda i: (0, 0)),
            ],
            out_specs=pl.BlockSpec((BB, R), lambda i: (i, 0)),
            out_shape=jax.ShapeDtypeStruct((BATCH // 2, R), jnp.int32),
            scratch_shapes=[pltpu.VMEM((R, DM), jnp.float32)],
            compiler_params=pltpu.CompilerParams(
                dimension_semantics=("arbitrary",),
            ),
            interpret=interpret,
        )(xr, cf, wf, rules_rep)
        halves.append(fb)
    f1b, f2b = halves
    k1b = _sc_select(f1b, interpret=interpret).reshape(BATCH // 2, 1)
    k2b = _sc_select(f2b, interpret=interpret).reshape(BATCH // 2, 1)

    out, norm, mask = pl.pallas_call(
        _tc2_body,
        grid=(GRID,),
        in_specs=[
            pl.BlockSpec((BB, D), lambda i: (i, 0)),
            pl.BlockSpec((BB, R), lambda i: (jnp.minimum(i, GRID // 2 - 1), 0)),
            pl.BlockSpec((BB, R),
                         lambda i: (jnp.maximum(i - GRID // 2, 0), 0)),
            pl.BlockSpec((BB, 1), lambda i: (jnp.minimum(i, GRID // 2 - 1), 0)),
            pl.BlockSpec((BB, 1),
                         lambda i: (jnp.maximum(i - GRID // 2, 0), 0)),
            pl.BlockSpec((R, JC), lambda i: (0, 0)),
        ],
        out_specs=[
            pl.BlockSpec((BB, C), lambda i: (i, 0)),
            pl.BlockSpec((BB, R), lambda i: (i, 0)),
            pl.BlockSpec((BB, R), lambda i: (i, 0)),
        ],
        out_shape=[
            jax.ShapeDtypeStruct((BATCH, C), jnp.float32),
            jax.ShapeDtypeStruct((BATCH, R), jnp.float32),
            jax.ShapeDtypeStruct((BATCH, R), jnp.float32),
        ],
        scratch_shapes=[pltpu.VMEM((R, C), jnp.float32)],
        compiler_params=pltpu.CompilerParams(
            dimension_semantics=("arbitrary",),
        ),
        interpret=interpret,
    )(x, f1b, f2b, k1b, k2b, cons2)
    return (out, norm, mask)


# speedup vs baseline: 1.0179x; 1.0179x over previous
import functools

import jax
import jax.numpy as jnp
from jax import lax
from jax.experimental import pallas as pl
from jax.experimental.pallas import tpu as pltpu
from jax.experimental.pallas import tpu_sc as plsc

BATCH = 1024
D = 26
M = 4
R = 2048
C = 16
DM = D * M
JC = (D + 1) * C
K = max(1, int(0.1 * R))

BB = 256
GRID = BATCH // BB

NC = 2
NS = 16
NW = NC * NS
RW = BATCH // NW
RV = R // 16


def _tc1_body(xr_ref, cf_ref, wf_ref, rules_ref, firing_ref, s_scr):
    @pl.when(pl.program_id(0) == 0)
    def _():
        mcol = lax.broadcasted_iota(jnp.int32, (R, DM), 1) % M
        s_scr[...] = (rules_ref[...] == mcol).astype(jnp.float32)

    wf = wf_ref[...]
    inv = 1.0 / (2.0 * wf * wf)
    dq = xr_ref[...] - cf_ref[...]
    q = dq * dq * inv
    logits = lax.dot_general(q, s_scr[...], (((1,), (1,)), ((), ())),
                             preferred_element_type=jnp.float32,
                             precision=lax.Precision.HIGHEST)
    firing_ref[...] = lax.bitcast_convert_type(jnp.exp(-logits), jnp.int32)


def _sel(vec, idx):
    lane = jnp.arange(16, dtype=jnp.int32)
    return jnp.sum(jnp.where(lane == idx, vec, 0))


def _sc_body(firing_hbm, kth_hbm, fbuf, hist, histc, cand, kbuf):
    wid = lax.axis_index("s") * NC + lax.axis_index("c")
    base = wid * RW
    pltpu.sync_copy(firing_hbm.at[pl.ds(base, RW)], fbuf)

    lane = jnp.arange(16, dtype=jnp.int32)
    zeros16 = jnp.zeros((16,), jnp.int32)
    ones16 = jnp.ones((16,), jnp.int32)

    @plsc.parallel_loop(0, R // 16, unroll=8)
    def _zero(i):
        hist[pl.ds(i * 16, 16)] = zeros16

    def row_body(i, _):
        lbase = lane * 128

        @plsc.parallel_loop(0, RV, unroll=8)
        def _hist(j):
            u = fbuf[i, pl.ds(j * 16, 16)]
            b = lax.shift_right_logical(u, 23)
            plsc.addupdate_scatter(hist, [lbase + b], ones16)

        t_vec = zeros16
        for c in range(8):
            acc = zeros16
            for l in range(16):
                off = l * 128 + c * 16
                acc = acc + hist[pl.ds(off, 16)]
                hist[pl.ds(off, 16)] = zeros16
            histc[pl.ds(c * 16, 16)] = acc
            t_vec = jnp.where(lane == 8 + c, jnp.sum(acc), t_vec)

        tr = lax.rev(t_vec, (0,))
        ct = plsc.cumsum(tr)
        a_exc = ct - tr
        i_star = plsc.all_reduce_ffs(ct >= K)
        i_s = _sel(lane, i_star)
        c_star = 7 - i_s
        cum_prev = _sel(a_exc, i_star)
        acc_star = histc[pl.ds(c_star * 16, 16)]
        racc = lax.rev(acc_star, (0,))
        pcl = plsc.cumsum(racc)
        j_star = plsc.all_reduce_ffs(cum_prev + pcl >= K)
        j_s = _sel(lane, j_star)
        e1 = c_star * 16 + 15 - j_s
        count_ge = cum_prev + _sel(pcl, j_star)
        m_e1 = _sel(racc, j_star)
        rem = K - (count_ge - m_e1)

        @plsc.parallel_loop(0, RV, unroll=4, carry=zeros16)
        def offv(j, ov):
            u = fbuf[i, pl.ds(j * 16, 16)]
            msk = lax.shift_right_logical(u, 23) == e1
            mi = msk.astype(jnp.int32)
            pos = plsc.cumsum(mi) - mi
            plsc.store_scatter(cand, [ov + pos], u, mask=msk)
            return ov + plsc.all_reduce_population_count(msk)
        m = jnp.sum(jnp.where(lane == 0, offv, 0))
        cand[pl.ds(m, 16)] = zeros16
        nv = lax.shift_right_logical(m + 15, 4)

        ebits = lax.shift_left(e1, 23)

        def bit_body(bi, t):
            t_try = jnp.bitwise_or(t, lax.shift_left(jnp.int32(1), 22 - bi))
            full = jnp.bitwise_or(ebits, t_try)

            def cnt_body(j, cv):
                u = cand[pl.ds(j * 16, 16)]
                return cv + (u >= full).astype(jnp.int32)

            cv = lax.fori_loop(0, nv, cnt_body, zeros16)
            return jnp.where(jnp.sum(cv) >= rem, t_try, t)

        t_fin = lax.fori_loop(0, 23, bit_body, jnp.int32(0))
        kf = jnp.bitwise_or(ebits, t_fin)
        plsc.store_scatter(kbuf, [jnp.full((16,), i, jnp.int32)],
                           jnp.full((16,), kf, jnp.int32),
                           mask=lane == 0)
        return 0

    lax.fori_loop(0, RW, row_body, 0)
    pltpu.sync_copy(kbuf, kth_hbm.at[pl.ds(base, RW)])


def _sc_select(firing_bits, interpret=False):
    mesh = plsc.VectorSubcoreMesh(core_axis_name="c", subcore_axis_name="s",
                                  num_cores=NC, num_subcores=NS)
    return pl.kernel(
        _sc_body,
        out_type=jax.ShapeDtypeStruct((BATCH,), jnp.int32),
        mesh=mesh,
        interpret=interpret,
        compiler_params=pltpu.CompilerParams(needs_layout_passes=False),
        scratch_types=[
            pltpu.VMEM((RW, R), jnp.int32),
            pltpu.VMEM((R,), jnp.int32),
            pltpu.VMEM((128,), jnp.int32),
            pltpu.VMEM((R + 32,), jnp.int32),
            pltpu.VMEM((RW,), jnp.int32),
        ],
    )(firing_bits)


def _tc2_body(x_ref, firing_ref, kth_ref, cons_ref,
              out_ref, norm_ref, mask_ref, cs_scr):
    @pl.when(pl.program_id(0) == 0)
    def _():
        p = (lax.broadcasted_iota(jnp.int32, (JC, C), 0) % C
             == lax.broadcasted_iota(jnp.int32, (JC, C), 1)).astype(jnp.float32)
        cs_scr[...] = lax.dot_general(cons_ref[...], p,
                                      (((1,), (0,)), ((), ())),
                                      preferred_element_type=jnp.float32,
                                      precision=lax.Precision.DEFAULT)

    fi = firing_ref[...]
    firing = lax.bitcast_convert_type(fi, jnp.float32)
    maskf = (fi >= kth_ref[...]).astype(jnp.float32)
    fm = firing * maskf
    denom = jnp.sum(fm, axis=1, keepdims=True) + 1e-9
    normv = fm / denom
    s_ext = jnp.sum(x_ref[...], axis=1, keepdims=True) + 1.0
    outv = s_ext * lax.dot_general(normv, cs_scr[...],
                                   (((1,), (0,)), ((), ())),
                                   preferred_element_type=jnp.float32,
                                   precision=lax.Precision.DEFAULT)
    out_ref[...] = outv
    norm_ref[...] = normv
    mask_ref[...] = maskf


@functools.partial(jax.jit, static_argnames=("interpret",))
def kernel(x, centers, widths, consequents, rules, interpret=False):
    x = x.astype(jnp.float32)
    xr = jnp.repeat(x, M, axis=1)
    cf = centers.astype(jnp.float32).reshape(1, DM)
    wf = widths.astype(jnp.float32).reshape(1, DM)
    rules_rep = jnp.repeat(rules.astype(jnp.int32), M, axis=1)
    cons2 = consequents.astype(jnp.float32).reshape(R, JC)

    firing_bits = pl.pallas_call(
        _tc1_body,
        grid=(GRID,),
        in_specs=[
            pl.BlockSpec((BB, DM), lambda i: (i, 0)),
            pl.BlockSpec((1, DM), lambda i: (0, 0)),
            pl.BlockSpec((1, DM), lambda i: (0, 0)),
            pl.BlockSpec((R, DM), lambda i: (0, 0)),
        ],
        out_specs=pl.BlockSpec((BB, R), lambda i: (i, 0)),
        out_shape=jax.ShapeDtypeStruct((BATCH, R), jnp.int32),
        scratch_shapes=[pltpu.VMEM((R, DM), jnp.float32)],
        compiler_params=pltpu.CompilerParams(
            dimension_semantics=("arbitrary",),
        ),
        interpret=interpret,
    )(xr, cf, wf, rules_rep)

    kth_bits = _sc_select(firing_bits, interpret=interpret).reshape(BATCH, 1)

    out, norm, mask = pl.pallas_call(
        _tc2_body,
        grid=(GRID,),
        in_specs=[
            pl.BlockSpec((BB, D), lambda i: (i, 0)),
            pl.BlockSpec((BB, R), lambda i: (i, 0)),
            pl.BlockSpec((BB, 1), lambda i: (i, 0)),
            pl.BlockSpec((R, JC), lambda i: (0, 0)),
        ],
        out_specs=[
            pl.BlockSpec((BB, C), lambda i: (i, 0)),
            pl.BlockSpec((BB, R), lambda i: (i, 0)),
            pl.BlockSpec((BB, R), lambda i: (i, 0)),
        ],
        out_shape=[
            jax.ShapeDtypeStruct((BATCH, C), jnp.float32),
            jax.ShapeDtypeStruct((BATCH, R), jnp.float32),
            jax.ShapeDtypeStruct((BATCH, R), jnp.float32),
        ],
        scratch_shapes=[pltpu.VMEM((R, C), jnp.float32)],
        compiler_params=pltpu.CompilerParams(
            dimension_semantics=("arbitrary",),
        ),
        interpret=interpret,
    )(x, firing_bits, kth_bits, cons2)
    return (out, norm, mask)


# speedup vs baseline: 1.0185x; 1.0006x over previous
import functools

import jax
import jax.numpy as jnp
from jax import lax
from jax.experimental import pallas as pl
from jax.experimental.pallas import tpu as pltpu
from jax.experimental.pallas import tpu_sc as plsc

BATCH = 1024
D = 26
M = 4
R = 2048
C = 16
DM = D * M
JC = (D + 1) * C
K = max(1, int(0.1 * R))

BB = 256
GRID = BATCH // BB

NC = 2
NS = 16
NW = NC * NS
RW = BATCH // NW
RV = R // 16


def _tc1_body(xr_ref, cf_ref, wf_ref, rules_ref, firing_ref, s_scr):
    @pl.when(pl.program_id(0) == 0)
    def _():
        mcol = lax.broadcasted_iota(jnp.int32, (R, DM), 1) % M
        s_scr[...] = (rules_ref[...] == mcol).astype(jnp.float32)

    wf = wf_ref[...]
    inv = 1.0 / (2.0 * wf * wf)
    dq = xr_ref[...] - cf_ref[...]
    q = dq * dq * inv
    logits = lax.dot_general(q, s_scr[...], (((1,), (1,)), ((), ())),
                             preferred_element_type=jnp.float32,
                             precision=lax.Precision.HIGHEST)
    firing_ref[...] = lax.bitcast_convert_type(jnp.exp(-logits), jnp.int32)


def _sel(vec, idx):
    lane = jnp.arange(16, dtype=jnp.int32)
    return jnp.sum(jnp.where(lane == idx, vec, 0))


def _sc_body(firing_hbm, kth_hbm, fbuf, hist, histc, cand, kbuf):
    wid = lax.axis_index("s") * NC + lax.axis_index("c")
    base = wid * RW
    pltpu.sync_copy(firing_hbm.at[pl.ds(base, RW)], fbuf)

    lane = jnp.arange(16, dtype=jnp.int32)
    zeros16 = jnp.zeros((16,), jnp.int32)
    ones16 = jnp.ones((16,), jnp.int32)

    @plsc.parallel_loop(0, R // 16, unroll=8)
    def _zero(i):
        hist[pl.ds(i * 16, 16)] = zeros16

    def row_body(i, _):
        lbase = lane * 128

        @plsc.parallel_loop(0, RV, unroll=8)
        def _hist(j):
            u = fbuf[i, pl.ds(j * 16, 16)]
            b = lax.shift_right_logical(u, 23)
            plsc.addupdate_scatter(hist, [lbase + b], ones16)

        t_vec = zeros16
        for c in range(8):
            acc = zeros16
            for l in range(16):
                off = l * 128 + c * 16
                acc = acc + hist[pl.ds(off, 16)]
                hist[pl.ds(off, 16)] = zeros16
            histc[pl.ds(c * 16, 16)] = acc
            t_vec = jnp.where(lane == 8 + c, jnp.sum(acc), t_vec)

        tr = lax.rev(t_vec, (0,))
        ct = plsc.cumsum(tr)
        a_exc = ct - tr
        i_star = plsc.all_reduce_ffs(ct >= K)
        i_s = _sel(lane, i_star)
        c_star = 7 - i_s
        cum_prev = _sel(a_exc, i_star)
        acc_star = histc[pl.ds(c_star * 16, 16)]
        racc = lax.rev(acc_star, (0,))
        pcl = plsc.cumsum(racc)
        j_star = plsc.all_reduce_ffs(cum_prev + pcl >= K)
        j_s = _sel(lane, j_star)
        e1 = c_star * 16 + 15 - j_s
        count_ge = cum_prev + _sel(pcl, j_star)
        m_e1 = _sel(racc, j_star)
        rem = K - (count_ge - m_e1)

        @plsc.parallel_loop(0, RV, unroll=4, carry=zeros16)
        def offv(j, ov):
            u = fbuf[i, pl.ds(j * 16, 16)]
            msk = lax.shift_right_logical(u, 23) == e1
            mi = msk.astype(jnp.int32)
            pos = plsc.cumsum(mi) - mi
            plsc.store_scatter(cand, [ov + pos], u, mask=msk)
            return ov + plsc.all_reduce_population_count(msk)
        m = jnp.sum(jnp.where(lane == 0, offv, 0))
        cand[pl.ds(m, 16)] = zeros16
        nv = lax.shift_right_logical(m + 15, 4)

        ebits = lax.shift_left(e1, 23)

        def bit_body(bi, t):
            t_try = jnp.bitwise_or(t, lax.shift_left(jnp.int32(1), 22 - bi))
            full = jnp.bitwise_or(ebits, t_try)

            def cnt_body(j, cv):
                u = cand[pl.ds(j * 16, 16)]
                return cv + (u >= full).astype(jnp.int32)

            cv = lax.fori_loop(0, nv, cnt_body, zeros16)
            return jnp.where(jnp.sum(cv) >= rem, t_try, t)

        t_fin = lax.fori_loop(0, 23, bit_body, jnp.int32(0))
        kf = jnp.bitwise_or(ebits, t_fin)
        plsc.store_scatter(kbuf, [jnp.full((16,), i, jnp.int32)],
                           jnp.full((16,), kf, jnp.int32),
                           mask=lane == 0)
        return 0

    lax.fori_loop(0, RW, row_body, 0)
    pltpu.sync_copy(kbuf, kth_hbm.at[pl.ds(base, RW)])


def _sc_select(firing_bits):
    mesh = plsc.VectorSubcoreMesh(core_axis_name="c", subcore_axis_name="s",
                                  num_cores=NC, num_subcores=NS)
    return pl.kernel(
        _sc_body,
        out_type=jax.ShapeDtypeStruct((BATCH,), jnp.int32),
        mesh=mesh,
        compiler_params=pltpu.CompilerParams(needs_layout_passes=False),
        scratch_types=[
            pltpu.VMEM((RW, R), jnp.int32),
            pltpu.VMEM((R,), jnp.int32),
            pltpu.VMEM((128,), jnp.int32),
            pltpu.VMEM((R + 32,), jnp.int32),
            pltpu.VMEM((RW,), jnp.int32),
        ],
    )(firing_bits)


def _tc2_body(x_ref, firing_ref, kth_ref, cons_ref,
              out_ref, norm_ref, mask_ref, cs_scr):
    @pl.when(pl.program_id(0) == 0)
    def _():
        p = (lax.broadcasted_iota(jnp.int32, (JC, C), 0) % C
             == lax.broadcasted_iota(jnp.int32, (JC, C), 1)).astype(jnp.float32)
        cs_scr[...] = lax.dot_general(cons_ref[...], p,
                                      (((1,), (0,)), ((), ())),
                                      preferred_element_type=jnp.float32,
                                      precision=lax.Precision.DEFAULT)

    fi = firing_ref[...]
    firing = lax.bitcast_convert_type(fi, jnp.float32)
    maskf = (fi >= kth_ref[...]).astype(jnp.float32)
    fm = firing * maskf
    denom = jnp.sum(fm, axis=1, keepdims=True) + 1e-9
    normv = fm / denom
    s_ext = jnp.sum(x_ref[...], axis=1, keepdims=True) + 1.0
    outv = s_ext * lax.dot_general(normv, cs_scr[...],
                                   (((1,), (0,)), ((), ())),
                                   preferred_element_type=jnp.float32,
                                   precision=lax.Precision.DEFAULT)
    out_ref[...] = outv
    norm_ref[...] = normv
    mask_ref[...] = maskf


@jax.jit
def kernel(x, centers, widths, consequents, rules):
    x = x.astype(jnp.float32)
    xr = jnp.repeat(x, M, axis=1)
    cf = centers.astype(jnp.float32).reshape(1, DM)
    wf = widths.astype(jnp.float32).reshape(1, DM)
    rules_rep = jnp.repeat(rules.astype(jnp.int32), M, axis=1)
    cons2 = consequents.astype(jnp.float32).reshape(R, JC)

    firing_bits = pl.pallas_call(
        _tc1_body,
        grid=(GRID,),
        in_specs=[
            pl.BlockSpec((BB, DM), lambda i: (i, 0)),
            pl.BlockSpec((1, DM), lambda i: (0, 0)),
            pl.BlockSpec((1, DM), lambda i: (0, 0)),
            pl.BlockSpec((R, DM), lambda i: (0, 0)),
        ],
        out_specs=pl.BlockSpec((BB, R), lambda i: (i, 0)),
        out_shape=jax.ShapeDtypeStruct((BATCH, R), jnp.int32),
        scratch_shapes=[pltpu.VMEM((R, DM), jnp.float32)],
        compiler_params=pltpu.CompilerParams(
            dimension_semantics=("arbitrary",),
        ),
    )(xr, cf, wf, rules_rep)

    kth_bits = _sc_select(firing_bits).reshape(BATCH, 1)

    out, norm, mask = pl.pallas_call(
        _tc2_body,
        grid=(GRID,),
        in_specs=[
            pl.BlockSpec((BB, D), lambda i: (i, 0)),
            pl.BlockSpec((BB, R), lambda i: (i, 0)),
            pl.BlockSpec((BB, 1), lambda i: (i, 0)),
            pl.BlockSpec((R, JC), lambda i: (0, 0)),
        ],
        out_specs=[
            pl.BlockSpec((BB, C), lambda i: (i, 0)),
            pl.BlockSpec((BB, R), lambda i: (i, 0)),
            pl.BlockSpec((BB, R), lambda i: (i, 0)),
        ],
        out_shape=[
            jax.ShapeDtypeStruct((BATCH, C), jnp.float32),
            jax.ShapeDtypeStruct((BATCH, R), jnp.float32),
            jax.ShapeDtypeStruct((BATCH, R), jnp.float32),
        ],
        scratch_shapes=[pltpu.VMEM((R, C), jnp.float32)],
        compiler_params=pltpu.CompilerParams(
            dimension_semantics=("arbitrary",),
        ),
    )(x, firing_bits, kth_bits, cons2)
    return (out, norm, mask)
